# baseline (speedup 1.0000x reference)
import jax
import jax.numpy as jnp
from jax.experimental import pallas as pl

_B, _S, _D_IN, _D_OUT = 8, 2048, 1024, 1024
_BM = 2048


def _mm_mask_kernel(x_ref, w_ref, b_ref, m_ref, o_ref):
    y = jnp.dot(x_ref[...].astype(jnp.bfloat16), w_ref[...].astype(jnp.bfloat16),
                preferred_element_type=jnp.float32)
    o_ref[...] = (y + b_ref[...]) * m_ref[...].astype(jnp.float32)


def kernel(x, mask, W, b):
    M = _B * _S
    x2 = x.reshape(M, _D_IN)
    mf = mask.reshape(M, 1)
    out = pl.pallas_call(
        _mm_mask_kernel,
        grid=(M // _BM,),
        in_specs=[
            pl.BlockSpec((_BM, _D_IN), lambda i: (i, 0)),
            pl.BlockSpec((_D_IN, _D_OUT), lambda i: (0, 0)),
            pl.BlockSpec((1, _D_OUT), lambda i: (0, 0)),
            pl.BlockSpec((_BM, 1), lambda i: (i, 0)),
        ],
        out_specs=pl.BlockSpec((_BM, _D_OUT), lambda i: (i, 0)),
        out_shape=jax.ShapeDtypeStruct((M, _D_OUT), jnp.float32),
    )(x2, W, b.reshape(1, _D_OUT), mf)
    return out.reshape(_B, _S, _D_OUT)
